# baseline (speedup 1.0000x reference)
import functools

import jax
import jax.numpy as jnp
from jax import lax
from jax.experimental import pallas as pl
from jax.experimental.pallas import tpu as pltpu
from jax.experimental.pallas import tpu_sc as plsc

NUM_EMB = 1_000_000
NUM_SUB = 8
CB_SIZE = 256
SUB_DIM = 16
EMB_DIM = NUM_SUB * SUB_DIM

N_TOKENS = 4096 * 50
NC, NS = 2, 16
NW = NC * NS
CHUNK = 128
PER_W = N_TOKENS // NW
N_CHUNKS = PER_W // CHUNK
ROWS = CHUNK * NUM_SUB


def _pq_body(ids_hbm, cb_hbm, codes_hbm, out_hbm,
             ids_v, codes_v, fidx_v, out_v, cb_sh, sem):
    cid = lax.axis_index("c")
    sid = lax.axis_index("s")
    wid = sid * NC + cid

    @pl.when(sid == 0)
    def _():
        pltpu.sync_copy(cb_hbm, cb_sh)

    plsc.subcore_barrier()

    def chunk_body(g, carry):
        iota = lax.iota(jnp.int32, 16)
        lane_div8 = iota // 8
        lane_mod8 = iota % 8
        bias = lane_mod8 * CB_SIZE
        base = (wid * N_CHUNKS + g) * CHUNK
        pltpu.sync_copy(ids_hbm.at[pl.ds(base, CHUNK)], ids_v)
        cps = [
            pltpu.async_copy(codes_hbm.at[s].at[ids_v], codes_v.at[s], sem)
            for s in range(NUM_SUB)
        ]
        for c in cps:
            c.wait()
        for i in range(ROWS // 16):
            col = lane_div8 + (2 * i)
            code = plsc.load_gather(codes_v, [lane_mod8, col])
            fidx_v[i // 8, pl.ds((i % 8) * 16, 16)] = code + bias
        copies = [
            pltpu.async_copy(cb_sh.at[fidx_v.at[j]],
                             out_v.at[pl.ds(j * CHUNK, CHUNK)], sem)
            for j in range(NUM_SUB)
        ]
        for c in copies:
            c.wait()
        pltpu.sync_copy(out_v, out_hbm.at[pl.ds(base * NUM_SUB, ROWS)])
        return carry

    lax.fori_loop(0, N_CHUNKS, chunk_body, 0)


@jax.jit
def _pq_lookup(ids_flat, cb_flat, codes):
    mesh = plsc.VectorSubcoreMesh(core_axis_name="c", subcore_axis_name="s")
    run = pl.kernel(
        _pq_body,
        out_type=jax.ShapeDtypeStruct((N_TOKENS * NUM_SUB, SUB_DIM),
                                      jnp.float32),
        mesh=mesh,
        compiler_params=pltpu.CompilerParams(use_tc_tiling_on_sc=False,
                                            needs_layout_passes=False),
        scratch_types=[
            pltpu.VMEM((CHUNK,), jnp.int32),
            pltpu.VMEM((NUM_SUB, CHUNK), jnp.int32),
            pltpu.VMEM((NUM_SUB, CHUNK), jnp.int32),
            pltpu.VMEM((ROWS, SUB_DIM), jnp.float32),
            pltpu.VMEM_SHARED((NUM_SUB * CB_SIZE, SUB_DIM), jnp.float32),
            pltpu.SemaphoreType.DMA,
        ],
    )
    return run(ids_flat, cb_flat, codes)


def kernel(input_ids, codebooks, codes):
    B, L = input_ids.shape
    ids_t = input_ids.T.reshape(-1).astype(jnp.int32)
    cb_flat = codebooks.reshape(NUM_SUB * CB_SIZE, SUB_DIM)
    codes_t = jnp.swapaxes(codes, 0, 1)
    out = _pq_lookup(ids_t, cb_flat, codes_t)
    return jnp.swapaxes(out.reshape(L, B, EMB_DIM), 0, 1)


# speedup vs baseline: 1.5983x; 1.5983x over previous
import jax
import jax.numpy as jnp
from jax import lax
from jax.experimental import pallas as pl
from jax.experimental.pallas import tpu as pltpu
from jax.experimental.pallas import tpu_sc as plsc

NUM_EMB = 1_000_000
NUM_SUB = 8
CB_SIZE = 256
SUB_DIM = 16
EMB_DIM = NUM_SUB * SUB_DIM

N_TOKENS = 4096 * 50
NC, NS = 2, 16
NW = NC * NS
CHUNK = 128
PER_W = N_TOKENS // NW
N_CHUNKS = PER_W // CHUNK
TOTAL_CHUNKS = NW * N_CHUNKS
ROWS = CHUNK * NUM_SUB


def _pq_body(ids_hbm, cb_hbm, codes_hbm, out_hbm,
             ids_v, codes_v, fidx_v, out_v, cb_sh, sg, sp, sw):
    cid = lax.axis_index("c")
    sid = lax.axis_index("s")
    wid = sid * NC + cid

    @pl.when(sid == 0)
    def _():
        pltpu.sync_copy(cb_hbm, cb_sh)

    plsc.subcore_barrier()

    iota = lax.iota(jnp.int32, 16)
    lane_div8 = iota // 8
    lane_mod8 = iota % 8
    bias = lane_mod8 * CB_SIZE

    def chunk_base(g):
        return ((wid * N_CHUNKS + g) % TOTAL_CHUNKS) * CHUNK

    def prefetch(g, b):
        pltpu.sync_copy(ids_hbm.at[pl.ds(chunk_base(g), CHUNK)], ids_v[b])
        pltpu.async_copy(codes_hbm.at[ids_v[b]], codes_v[b], sg[b])

    def expand(g, b):
        pltpu.make_async_copy(codes_hbm.at[ids_v[b]], codes_v[b],
                              sg[b]).wait()
        prefetch(g + 1, 1 - b)
        for i in range(ROWS // 16):
            row = lane_div8 + (2 * i)
            code = plsc.load_gather(codes_v[b], [row, lane_mod8])
            fidx_v[b][i // 8, pl.ds((i % 8) * 16, 16)] = code + bias
        @pl.when(g >= 2)
        def _():
            pltpu.make_async_copy(
                out_v[b],
                out_hbm.at[pl.ds(chunk_base(g - 2) * NUM_SUB, ROWS)],
                sw[b]).wait()
        copies = [
            pltpu.async_copy(cb_sh.at[fidx_v[b].at[j]],
                             out_v[b].at[pl.ds(j * CHUNK, CHUNK)], sp[b])
            for j in range(NUM_SUB)
        ]
        for c in copies:
            c.wait()
        pltpu.async_copy(out_v[b],
                         out_hbm.at[pl.ds(chunk_base(g) * NUM_SUB, ROWS)],
                         sw[b])

    prefetch(0, 0)

    def pair_body(g2, carry):
        expand(2 * g2, 0)
        expand(2 * g2 + 1, 1)
        return carry

    lax.fori_loop(0, N_CHUNKS // 2, pair_body, 0)

    pltpu.make_async_copy(
        out_v[0], out_hbm.at[pl.ds(chunk_base(N_CHUNKS - 2) * NUM_SUB, ROWS)],
        sw[0]).wait()
    pltpu.make_async_copy(
        out_v[1], out_hbm.at[pl.ds(chunk_base(N_CHUNKS - 1) * NUM_SUB, ROWS)],
        sw[1]).wait()
    pltpu.make_async_copy(codes_hbm.at[ids_v[0]], codes_v[0], sg[0]).wait()


@jax.jit
def _pq_lookup(ids_flat, cb_flat, codes):
    mesh = plsc.VectorSubcoreMesh(core_axis_name="c", subcore_axis_name="s")
    run = pl.kernel(
        _pq_body,
        out_type=jax.ShapeDtypeStruct((N_TOKENS * NUM_SUB, SUB_DIM),
                                      jnp.float32),
        mesh=mesh,
        compiler_params=pltpu.CompilerParams(use_tc_tiling_on_sc=False,
                                             needs_layout_passes=False),
        scratch_types=[
            [pltpu.VMEM((CHUNK,), jnp.int32)] * 2,
            [pltpu.VMEM((CHUNK, NUM_SUB), jnp.int32)] * 2,
            [pltpu.VMEM((NUM_SUB, CHUNK), jnp.int32)] * 2,
            [pltpu.VMEM((ROWS, SUB_DIM), jnp.float32)] * 2,
            pltpu.VMEM_SHARED((NUM_SUB * CB_SIZE, SUB_DIM), jnp.float32),
            [pltpu.SemaphoreType.DMA] * 2,
            [pltpu.SemaphoreType.DMA] * 2,
            [pltpu.SemaphoreType.DMA] * 2,
        ],
    )
    return run(ids_flat, cb_flat, codes)


def kernel(input_ids, codebooks, codes):
    B, L = input_ids.shape
    ids_t = input_ids.T.reshape(-1).astype(jnp.int32)
    cb_flat = codebooks.reshape(NUM_SUB * CB_SIZE, SUB_DIM)
    codes_lin = jax.lax.optimization_barrier(codes.reshape(-1))
    codes_2d = codes_lin.reshape(NUM_EMB, NUM_SUB)
    out = _pq_lookup(ids_t, cb_flat, codes_2d)
    return jnp.swapaxes(out.reshape(L, B, EMB_DIM), 0, 1)


# speedup vs baseline: 2.7609x; 1.7274x over previous
import jax
import jax.numpy as jnp
from jax import lax
from jax.experimental import pallas as pl
from jax.experimental.pallas import tpu as pltpu
from jax.experimental.pallas import tpu_sc as plsc

NUM_EMB = 1_000_000
NUM_SUB = 8
CB_SIZE = 256
SUB_DIM = 16
EMB_DIM = NUM_SUB * SUB_DIM

N_TOKENS = 4096 * 50
NC, NS = 2, 16
NW = NC * NS
CHUNK = 128
PER_W = N_TOKENS // NW
N_CHUNKS = PER_W // CHUNK
TOTAL_CHUNKS = NW * N_CHUNKS
ROWS = CHUNK * NUM_SUB


def _pq_body(ids_hbm, cb_hbm, c0, c1, c2, c3, c4, c5, c6, c7, out_hbm,
             ids_v, codes_v, fidx_v, out_v, cb_sh, sg, sp, sw):
    cs = (c0, c1, c2, c3, c4, c5, c6, c7)
    cid = lax.axis_index("c")
    sid = lax.axis_index("s")
    wid = sid * NC + cid

    @pl.when(sid == 0)
    def _():
        pltpu.sync_copy(cb_hbm, cb_sh)

    plsc.subcore_barrier()

    iota = lax.iota(jnp.int32, 16)
    lane_div8 = iota // 8
    lane_mod8 = iota % 8
    bias = lane_mod8 * CB_SIZE

    def chunk_base(g):
        return ((wid * N_CHUNKS + g) % TOTAL_CHUNKS) * CHUNK

    def prefetch(g, b):
        pltpu.sync_copy(ids_hbm.at[pl.ds(chunk_base(g), CHUNK)], ids_v[b])
        for s in range(NUM_SUB):
            pltpu.async_copy(cs[s].at[ids_v[b]], codes_v[b].at[s], sg[b])

    def expand(g, b):
        for s in range(NUM_SUB):
            pltpu.make_async_copy(cs[s].at[ids_v[b]], codes_v[b].at[s],
                                  sg[b]).wait()
        prefetch(g + 1, 1 - b)
        for i in range(ROWS // 16):
            col = lane_div8 + (2 * i)
            code = plsc.load_gather(codes_v[b], [lane_mod8, col])
            fidx_v[b][i // 8, pl.ds((i % 8) * 16, 16)] = code + bias
        @pl.when(g >= 2)
        def _():
            pltpu.make_async_copy(
                out_v[b],
                out_hbm.at[pl.ds(chunk_base(g - 2) * NUM_SUB, ROWS)],
                sw[b]).wait()
        copies = [
            pltpu.async_copy(cb_sh.at[fidx_v[b].at[j]],
                             out_v[b].at[pl.ds(j * CHUNK, CHUNK)], sp[b])
            for j in range(NUM_SUB)
        ]
        for c in copies:
            c.wait()
        pltpu.async_copy(out_v[b],
                         out_hbm.at[pl.ds(chunk_base(g) * NUM_SUB, ROWS)],
                         sw[b])

    prefetch(0, 0)

    def pair_body(g2, carry):
        expand(2 * g2, 0)
        expand(2 * g2 + 1, 1)
        return carry

    lax.fori_loop(0, N_CHUNKS // 2, pair_body, 0)

    pltpu.make_async_copy(
        out_v[0], out_hbm.at[pl.ds(chunk_base(N_CHUNKS - 2) * NUM_SUB, ROWS)],
        sw[0]).wait()
    pltpu.make_async_copy(
        out_v[1], out_hbm.at[pl.ds(chunk_base(N_CHUNKS - 1) * NUM_SUB, ROWS)],
        sw[1]).wait()
    for s in range(NUM_SUB):
        pltpu.make_async_copy(cs[s].at[ids_v[0]], codes_v[0].at[s],
                              sg[0]).wait()


@jax.jit
def _pq_lookup(ids_flat, cb_flat, *codes_cols):
    mesh = plsc.VectorSubcoreMesh(core_axis_name="c", subcore_axis_name="s")
    run = pl.kernel(
        _pq_body,
        out_type=jax.ShapeDtypeStruct((N_TOKENS * NUM_SUB, SUB_DIM),
                                      jnp.float32),
        mesh=mesh,
        compiler_params=pltpu.CompilerParams(use_tc_tiling_on_sc=False,
                                             needs_layout_passes=False),
        scratch_types=[
            [pltpu.VMEM((CHUNK,), jnp.int32)] * 2,
            [pltpu.VMEM((NUM_SUB, CHUNK), jnp.int32)] * 2,
            [pltpu.VMEM((NUM_SUB, CHUNK), jnp.int32)] * 2,
            [pltpu.VMEM((ROWS, SUB_DIM), jnp.float32)] * 2,
            pltpu.VMEM_SHARED((NUM_SUB * CB_SIZE, SUB_DIM), jnp.float32),
            [pltpu.SemaphoreType.DMA] * 2,
            [pltpu.SemaphoreType.DMA] * 2,
            [pltpu.SemaphoreType.DMA] * 2,
        ],
    )
    return run(ids_flat, cb_flat, *codes_cols)


def kernel(input_ids, codebooks, codes):
    B, L = input_ids.shape
    ids_t = input_ids.T.reshape(-1).astype(jnp.int32)
    cb_flat = codebooks.reshape(NUM_SUB * CB_SIZE, SUB_DIM)
    codes_cols = [codes[:, s] for s in range(NUM_SUB)]
    out = _pq_lookup(ids_t, cb_flat, *codes_cols)
    return jnp.swapaxes(out.reshape(L, B, EMB_DIM), 0, 1)


# speedup vs baseline: 4.1338x; 1.4973x over previous
import jax
import jax.numpy as jnp
from jax import lax
from jax.experimental import pallas as pl
from jax.experimental.pallas import tpu as pltpu
from jax.experimental.pallas import tpu_sc as plsc

NUM_EMB = 1_000_000
NUM_SUB = 8
CB_SIZE = 256
SUB_DIM = 16
EMB_DIM = NUM_SUB * SUB_DIM

N_TOKENS = 4096 * 50
NC, NS = 2, 16
NW = NC * NS
CHUNK = 128
PER_W = N_TOKENS // NW
N_CHUNKS = PER_W // CHUNK
TOTAL_CHUNKS = NW * N_CHUNKS
ROWS = CHUNK * NUM_SUB


def _pq_body(ids_hbm, cb_hbm, q0, q1, out_hbm,
             ids_v, codes_v, fidx_v, out_v, cb_sh, sg, sp, sw):
    qs = (q0, q1)
    cid = lax.axis_index("c")
    sid = lax.axis_index("s")
    wid = sid * NC + cid

    @pl.when(sid == 0)
    def _():
        pltpu.sync_copy(cb_hbm, cb_sh)

    plsc.subcore_barrier()

    iota = lax.iota(jnp.int32, 16)
    lane_div8 = iota // 8
    lane_mod8 = iota % 8
    bias = lane_mod8 * CB_SIZE
    lane_quad = lane_mod8 // 4
    lane_shift = (lane_mod8 % 4) * 8

    def chunk_base(g):
        return ((wid * N_CHUNKS + g) % TOTAL_CHUNKS) * CHUNK

    def prefetch(g, b):
        pltpu.sync_copy(ids_hbm.at[pl.ds(chunk_base(g), CHUNK)], ids_v[b])
        for q in range(2):
            pltpu.async_copy(qs[q].at[ids_v[b]], codes_v[b].at[q], sg[b])

    def expand(g, b):
        for q in range(2):
            pltpu.make_async_copy(qs[q].at[ids_v[b]], codes_v[b].at[q],
                                  sg[b]).wait()
        prefetch(g + 1, 1 - b)
        for i in range(ROWS // 16):
            col = lane_div8 + (2 * i)
            quad = plsc.load_gather(codes_v[b], [lane_quad, col])
            code = (quad >> lane_shift) & 0xFF
            fidx_v[b][i // 8, pl.ds((i % 8) * 16, 16)] = code + bias
        @pl.when(g >= 2)
        def _():
            pltpu.make_async_copy(
                out_v[b],
                out_hbm.at[pl.ds(chunk_base(g - 2) * NUM_SUB, ROWS)],
                sw[b]).wait()
        copies = [
            pltpu.async_copy(cb_sh.at[fidx_v[b].at[j]],
                             out_v[b].at[pl.ds(j * CHUNK, CHUNK)], sp[b])
            for j in range(NUM_SUB)
        ]
        for c in copies:
            c.wait()
        pltpu.async_copy(out_v[b],
                         out_hbm.at[pl.ds(chunk_base(g) * NUM_SUB, ROWS)],
                         sw[b])

    prefetch(0, 0)

    def pair_body(g2, carry):
        expand(2 * g2, 0)
        expand(2 * g2 + 1, 1)
        return carry

    lax.fori_loop(0, N_CHUNKS // 2, pair_body, 0)

    pltpu.make_async_copy(
        out_v[0], out_hbm.at[pl.ds(chunk_base(N_CHUNKS - 2) * NUM_SUB, ROWS)],
        sw[0]).wait()
    pltpu.make_async_copy(
        out_v[1], out_hbm.at[pl.ds(chunk_base(N_CHUNKS - 1) * NUM_SUB, ROWS)],
        sw[1]).wait()
    for q in range(2):
        pltpu.make_async_copy(qs[q].at[ids_v[0]], codes_v[0].at[q],
                              sg[0]).wait()


@jax.jit
def _pq_lookup(ids_flat, cb_flat, q0, q1):
    mesh = plsc.VectorSubcoreMesh(core_axis_name="c", subcore_axis_name="s")
    run = pl.kernel(
        _pq_body,
        out_type=jax.ShapeDtypeStruct((N_TOKENS * NUM_SUB, SUB_DIM),
                                      jnp.float32),
        mesh=mesh,
        compiler_params=pltpu.CompilerParams(use_tc_tiling_on_sc=False,
                                             needs_layout_passes=False),
        scratch_types=[
            [pltpu.VMEM((CHUNK,), jnp.int32)] * 2,
            [pltpu.VMEM((2, CHUNK), jnp.int32)] * 2,
            [pltpu.VMEM((NUM_SUB, CHUNK), jnp.int32)] * 2,
            [pltpu.VMEM((ROWS, SUB_DIM), jnp.float32)] * 2,
            pltpu.VMEM_SHARED((NUM_SUB * CB_SIZE, SUB_DIM), jnp.float32),
            [pltpu.SemaphoreType.DMA] * 2,
            [pltpu.SemaphoreType.DMA] * 2,
            [pltpu.SemaphoreType.DMA] * 2,
        ],
    )
    return run(ids_flat, cb_flat, q0, q1)


def kernel(input_ids, codebooks, codes):
    B, L = input_ids.shape
    ids_t = input_ids.T.reshape(-1).astype(jnp.int32)
    cb_flat = codebooks.reshape(NUM_SUB * CB_SIZE, SUB_DIM)
    def pack4(a, b, c, d):
        return a | (b << 8) | (c << 16) | (d << 24)
    q0 = pack4(codes[:, 0], codes[:, 1], codes[:, 2], codes[:, 3])
    q1 = pack4(codes[:, 4], codes[:, 5], codes[:, 6], codes[:, 7])
    out = _pq_lookup(ids_t, cb_flat, q0, q1)
    return jnp.swapaxes(out.reshape(L, B, EMB_DIM), 0, 1)


# speedup vs baseline: 4.3300x; 1.0474x over previous
import jax
import jax.numpy as jnp
from jax import lax
from jax.experimental import pallas as pl
from jax.experimental.pallas import tpu as pltpu
from jax.experimental.pallas import tpu_sc as plsc

NUM_EMB = 1_000_000
NUM_SUB = 8
CB_SIZE = 256
SUB_DIM = 16
EMB_DIM = NUM_SUB * SUB_DIM

N_TOKENS = 4096 * 50
NC, NS = 2, 16
NW = NC * NS
CHUNK = 128
PER_W = N_TOKENS // NW
N_CHUNKS = PER_W // CHUNK
TOTAL_CHUNKS = NW * N_CHUNKS
ROWS = CHUNK * NUM_SUB


def _pq_body(ids_hbm, cb_hbm, q0, q1, out_hbm,
             ids_v, codes_v, fidx_v, out_v, cb_sh, sg, sp, sw):
    qs = (q0, q1)
    cid = lax.axis_index("c")
    sid = lax.axis_index("s")
    wid = sid * NC + cid

    @pl.when(sid == 0)
    def _():
        pltpu.sync_copy(cb_hbm, cb_sh)

    plsc.subcore_barrier()

    iota = lax.iota(jnp.int32, 16)
    lane_div8 = iota // 8
    lane_mod8 = iota % 8
    bias = lane_mod8 * CB_SIZE
    lane_quad = lane_mod8 // 4
    lane_shift = (lane_mod8 % 4) * 8

    def chunk_base(g):
        return ((wid * N_CHUNKS + g) % TOTAL_CHUNKS) * CHUNK

    def prefetch(g, b):
        pltpu.sync_copy(ids_hbm.at[pl.ds(chunk_base(g), CHUNK)], ids_v[b])
        for q in range(2):
            pltpu.async_copy(qs[q].at[ids_v[b]], codes_v[b].at[q], sg[b])

    def expand(g, b):
        for q in range(2):
            pltpu.make_async_copy(qs[q].at[ids_v[b]], codes_v[b].at[q],
                                  sg[b]).wait()
        prefetch(g + 1, 1 - b)
        for i in range(ROWS // 16):
            col = lane_div8 + (2 * i)
            quad = plsc.load_gather(codes_v[b], [lane_quad, col])
            code = (quad >> lane_shift) & 0xFF
            fidx_v[b][i // 8, pl.ds((i % 8) * 16, 16)] = code + bias
        @pl.when(g >= 2)
        def _():
            pltpu.make_async_copy(
                out_v[b],
                out_hbm.at[pl.ds(chunk_base(g - 2) * NUM_SUB, ROWS)],
                sw[b]).wait()
        copies = [
            pltpu.async_copy(cb_sh.at[fidx_v[b].at[j]],
                             out_v[b].at[pl.ds(j * CHUNK, CHUNK)], sp[b])
            for j in range(NUM_SUB)
        ]
        for c in copies:
            c.wait()
        pltpu.async_copy(out_v[b],
                         out_hbm.at[pl.ds(chunk_base(g) * NUM_SUB, ROWS)],
                         sw[b])

    prefetch(0, 0)

    def pair_body(g2, carry):
        expand(2 * g2, 0)
        expand(2 * g2 + 1, 1)
        return carry

    lax.fori_loop(0, N_CHUNKS // 2, pair_body, 0)

    pltpu.make_async_copy(
        out_v[0], out_hbm.at[pl.ds(chunk_base(N_CHUNKS - 2) * NUM_SUB, ROWS)],
        sw[0]).wait()
    pltpu.make_async_copy(
        out_v[1], out_hbm.at[pl.ds(chunk_base(N_CHUNKS - 1) * NUM_SUB, ROWS)],
        sw[1]).wait()
    for q in range(2):
        pltpu.make_async_copy(qs[q].at[ids_v[0]], codes_v[0].at[q],
                              sg[0]).wait()


@jax.jit
def _pq_lookup(ids_flat, cb_flat, q0, q1):
    mesh = plsc.VectorSubcoreMesh(core_axis_name="c", subcore_axis_name="s")
    run = pl.kernel(
        _pq_body,
        out_type=jax.ShapeDtypeStruct((N_TOKENS * NUM_SUB, SUB_DIM),
                                      jnp.float32),
        mesh=mesh,
        compiler_params=pltpu.CompilerParams(use_tc_tiling_on_sc=False,
                                             needs_layout_passes=False),
        scratch_types=[
            [pltpu.VMEM((CHUNK,), jnp.int32)] * 2,
            [pltpu.VMEM((2, CHUNK), jnp.int32)] * 2,
            [pltpu.VMEM((NUM_SUB, CHUNK), jnp.int32)] * 2,
            [pltpu.VMEM((ROWS, SUB_DIM), jnp.float32)] * 2,
            pltpu.VMEM_SHARED((NUM_SUB * CB_SIZE, SUB_DIM), jnp.float32),
            [pltpu.SemaphoreType.DMA] * 2,
            [pltpu.SemaphoreType.DMA] * 2,
            [pltpu.SemaphoreType.DMA] * 2,
        ],
    )
    return run(ids_flat, cb_flat, q0, q1)


def kernel(input_ids, codebooks, codes):
    B, L = input_ids.shape
    ids_t = input_ids.T.reshape(-1).astype(jnp.int32)
    cb_flat = codebooks.reshape(NUM_SUB * CB_SIZE, SUB_DIM)
    w = jnp.zeros((NUM_SUB, 4), jnp.float32)
    w = w.at[0, 0].set(1.0).at[1, 0].set(256.0)
    w = w.at[2, 1].set(1.0).at[3, 1].set(256.0)
    w = w.at[4, 2].set(1.0).at[5, 2].set(256.0)
    w = w.at[6, 3].set(1.0).at[7, 3].set(256.0)
    halves = jax.lax.dot(codes.astype(jnp.float32), w,
                         precision=jax.lax.Precision.HIGHEST)
    h = halves.astype(jnp.int32)
    q0 = h[:, 0] | (h[:, 1] << 16)
    q1 = h[:, 2] | (h[:, 3] << 16)
    out = _pq_lookup(ids_t, cb_flat, q0, q1)
    return jnp.swapaxes(out.reshape(L, B, EMB_DIM), 0, 1)
